# Initial kernel scaffold; baseline (speedup 1.0000x reference)
import jax
import jax.numpy as jnp
from jax.experimental import pallas as pl


def kernel(x_list, hg, W1, b1, W2, b2):
    raise NotImplementedError("write your pallas kernel here")


# speedup vs baseline: 1.0053x; 1.0053x over previous
import functools
import jax
import jax.numpy as jnp
from jax.experimental import pallas as pl
from jax.experimental.pallas import tpu as pltpu


def _matmul_body(x_ref, w_ref, b_ref, o_ref):
    o_ref[...] = (
        jnp.dot(x_ref[...], w_ref[...], preferred_element_type=jnp.float32)
        + b_ref[...]
    )


def _project(X, W, b, block_rows=1000):
    N, d = X.shape
    h = W.shape[1]
    grid = (N // block_rows,)
    return pl.pallas_call(
        _matmul_body,
        grid=grid,
        in_specs=[
            pl.BlockSpec((block_rows, d), lambda i: (i, 0)),
            pl.BlockSpec((d, h), lambda i: (0, 0)),
            pl.BlockSpec((h,), lambda i: (0,)),
        ],
        out_specs=pl.BlockSpec((block_rows, h), lambda i: (i, 0)),
        out_shape=jax.ShapeDtypeStruct((N, h), jnp.float32),
    )(X, W, b)


def _conv_rest(X, hg, is_last):
    E, s = hg.shape
    N = X.shape[0]
    feats = X[hg]
    sq = jnp.sum(feats * feats, axis=-1)
    gram = jnp.einsum('esd,etd->est', feats, feats)
    d2 = sq[:, :, None] + sq[:, None, :] - 2.0 * gram
    amax = jnp.argmax(d2.reshape(E, s * s), axis=1)
    u = jnp.take_along_axis(hg, (amax // s)[:, None], axis=1)[:, 0]
    v = jnp.take_along_axis(hg, (amax % s)[:, None], axis=1)[:, 0]
    loop = jnp.arange(N, dtype=u.dtype)
    src = jnp.concatenate([u, v, loop])
    dst = jnp.concatenate([v, u, loop])
    deg = jax.ops.segment_sum(jnp.ones((src.shape[0],), dtype=X.dtype), dst,
                              num_segments=N)
    dinv = jnp.where(deg > 0, 1.0 / jnp.sqrt(deg), 0.0)
    coef = dinv[src] * dinv[dst]
    out = jax.ops.segment_sum(X[src] * coef[:, None], dst, num_segments=N)
    if not is_last:
        out = jax.nn.relu(out)
    return out


def kernel(x_list, hg, W1, b1, W2, b2):
    hidden = []
    for k in range(x_list.shape[0]):
        Xp = _project(x_list[k], W1, b1)
        hidden.append(_conv_rest(Xp, hg, False))
    h = jnp.concatenate(hidden, axis=-1)
    Hp = _project(h, W2, b2)
    return _conv_rest(Hp, hg, True)


# speedup vs baseline: 1.1360x; 1.1301x over previous
import functools
import jax
import jax.numpy as jnp
from jax import lax
from jax.experimental import pallas as pl
from jax.experimental.pallas import tpu as pltpu, tpu_sc as plsc

N = 10000
NPAD = 10240
E = 40000
NW = 32


def _matmul_body(x_ref, w_ref, b_ref, o_ref):
    o_ref[...] = (
        jnp.dot(x_ref[...], w_ref[...], preferred_element_type=jnp.float32)
        + b_ref[...]
    )


def _project(X, W, b, block_rows=1024):
    Nr, d = X.shape
    h = W.shape[1]
    return pl.pallas_call(
        _matmul_body,
        grid=(Nr // block_rows,),
        in_specs=[
            pl.BlockSpec((block_rows, d), lambda i: (i, 0)),
            pl.BlockSpec((d, h), lambda i: (0, 0)),
            pl.BlockSpec((h,), lambda i: (0,)),
        ],
        out_specs=pl.BlockSpec((block_rows, h), lambda i: (i, 0)),
        out_shape=jax.ShapeDtypeStruct((Nr, h), jnp.float32),
    )(X, W, b)


def _make_gather(n_idx, width, table_rows):
    per_w = ((n_idx + NW - 1) // NW + 127) // 128 * 128
    C = 128
    mesh = plsc.VectorSubcoreMesh(core_axis_name="c", subcore_axis_name="s")

    @functools.partial(
        pl.kernel,
        out_type=jax.ShapeDtypeStruct((n_idx, width), jnp.float32),
        mesh=mesh,
        scratch_types=[
            pltpu.VMEM((C,), jnp.int32),
            pltpu.VMEM((C, width), jnp.float32),
            pltpu.SemaphoreType.DMA,
        ],
    )
    def gather_k(table_hbm, idx_hbm, out_hbm, idx_v, rows_v, sem):
        wid = lax.axis_index("s") * 2 + lax.axis_index("c")
        base = wid * per_w
        ntodo = jnp.clip(n_idx - base, 0, per_w)
        nchunk = ntodo // C

        def body(j, _):
            off = base + j * C
            pltpu.sync_copy(idx_hbm.at[pl.ds(off, C)], idx_v)
            pltpu.async_copy(table_hbm.at[idx_v], rows_v, sem).wait()
            pltpu.sync_copy(rows_v, out_hbm.at[pl.ds(off, C)])
            return 0

        lax.fori_loop(0, nchunk, body, 0)

    return gather_k


_gather_feats_256 = _make_gather(4 * E, 256, NPAD)
_gather_feats_128 = _make_gather(4 * E, 128, NPAD)


def _conv_rest(X, Xpad, hg, feats, is_last):
    E_, s = hg.shape
    sq = jnp.sum(feats * feats, axis=-1)
    gram = jnp.einsum('esd,etd->est', feats, feats)
    d2 = sq[:, :, None] + sq[:, None, :] - 2.0 * gram
    amax = jnp.argmax(d2.reshape(E_, s * s), axis=1)
    u = jnp.take_along_axis(hg, (amax // s)[:, None], axis=1)[:, 0]
    v = jnp.take_along_axis(hg, (amax % s)[:, None], axis=1)[:, 0]
    loop = jnp.arange(N, dtype=u.dtype)
    src = jnp.concatenate([u, v, loop])
    dst = jnp.concatenate([v, u, loop])
    deg = jax.ops.segment_sum(jnp.ones((src.shape[0],), dtype=X.dtype), dst,
                              num_segments=N)
    dinv = jnp.where(deg > 0, 1.0 / jnp.sqrt(deg), 0.0)
    coef = dinv[src] * dinv[dst]
    out = jax.ops.segment_sum(X[src] * coef[:, None], dst, num_segments=N)
    if not is_last:
        out = jax.nn.relu(out)
    return out


def kernel(x_list, hg, W1, b1, W2, b2):
    hg_flat = hg.reshape(-1)
    hidden = []
    for k in range(x_list.shape[0]):
        xp = jnp.pad(x_list[k], ((0, NPAD - N), (0, 0)))
        Xp = _project(xp, W1, b1)
        feats = _gather_feats_256(Xp, hg_flat).reshape(E, 4, 256)
        hidden.append(_conv_rest(Xp[:N], Xp, hg, feats, False))
    h = jnp.concatenate(hidden, axis=-1)
    hpad = jnp.pad(h, ((0, NPAD - N), (0, 0)))
    Hp = _project(hpad, W2, b2)
    Hp128 = jnp.pad(Hp, ((0, 0), (0, 64)))
    feats2 = _gather_feats_128(Hp128, hg_flat)[:, :64].reshape(E, 4, 64)
    return _conv_rest(Hp[:N], Hp, hg, feats2, True)


# speedup vs baseline: 4.1216x; 3.6280x over previous
import functools
import jax
import jax.numpy as jnp
from jax import lax
from jax.experimental import pallas as pl
from jax.experimental.pallas import tpu as pltpu, tpu_sc as plsc

N = 10000
NPAD = 10240
E = 40000
EPAD = 40960
NW = 32


def _matmul_body(x_ref, w_ref, b_ref, o_ref):
    o_ref[...] = (
        jnp.dot(x_ref[...], w_ref[...], preferred_element_type=jnp.float32)
        + b_ref[...]
    )


def _project(X, W, b, block_rows=1024):
    Nr, d = X.shape
    h = W.shape[1]
    return pl.pallas_call(
        _matmul_body,
        grid=(Nr // block_rows,),
        in_specs=[
            pl.BlockSpec((block_rows, d), lambda i: (i, 0)),
            pl.BlockSpec((d, h), lambda i: (0, 0)),
            pl.BlockSpec((h,), lambda i: (0,)),
        ],
        out_specs=pl.BlockSpec((block_rows, h), lambda i: (i, 0)),
        out_shape=jax.ShapeDtypeStruct((Nr, h), jnp.float32),
    )(X, W, b)


def _make_gather(n_idx, width, table_rows):
    per_w = ((n_idx + NW - 1) // NW + 127) // 128 * 128
    C = 128
    mesh = plsc.VectorSubcoreMesh(core_axis_name="c", subcore_axis_name="s")

    @functools.partial(
        pl.kernel,
        out_type=jax.ShapeDtypeStruct((n_idx, width), jnp.float32),
        mesh=mesh,
        scratch_types=[
            pltpu.VMEM((C,), jnp.int32),
            pltpu.VMEM((C, width), jnp.float32),
            pltpu.SemaphoreType.DMA,
        ],
    )
    def gather_k(table_hbm, idx_hbm, out_hbm, idx_v, rows_v, sem):
        wid = lax.axis_index("s") * 2 + lax.axis_index("c")
        base = wid * per_w
        ntodo = jnp.clip(n_idx - base, 0, per_w)
        nchunk = ntodo // C

        def body(j, _):
            off = base + j * C
            pltpu.sync_copy(idx_hbm.at[pl.ds(off, C)], idx_v)
            pltpu.async_copy(table_hbm.at[idx_v], rows_v, sem).wait()
            pltpu.sync_copy(rows_v, out_hbm.at[pl.ds(off, C)])
            return 0

        lax.fori_loop(0, nchunk, body, 0)

    return gather_k


_gather_feats_256 = _make_gather(4 * E, 256, NPAD)
_gather_feats_128 = _make_gather(4 * E, 128, NPAD)


def _make_msg(split_edges):
    C = 128
    mesh = plsc.VectorSubcoreMesh(core_axis_name="c", subcore_axis_name="s")

    @functools.partial(
        pl.kernel,
        out_type=jax.ShapeDtypeStruct((2, NPAD, 128), jnp.float32),
        mesh=mesh,
        scratch_types=[
            pltpu.VMEM((C,), jnp.int32),
            pltpu.VMEM((C,), jnp.int32),
            pltpu.VMEM((C,), jnp.int32),
            pltpu.VMEM((C,), jnp.int32),
            pltpu.VMEM((C, 128), jnp.float32),
            pltpu.VMEM((C, 128), jnp.float32),
            pltpu.VMEM((64, 128), jnp.float32),
            pltpu.VMEM_SHARED((NPAD, 128), jnp.float32),
            pltpu.SemaphoreType.DMA,
            pltpu.SemaphoreType.DMA,
        ],
    )
    def msg_k(y_hbm, u_hbm, v_hbm, out_hbm, uraw_v, vraw_v, ug_v, vg_v,
              rowsA_v, rowsB_v, zero_v, acc_sh, semA, semB):
        cid = lax.axis_index("c")
        sid = lax.axis_index("s")

        def zb(i, _):
            zero_v[i // 8, pl.ds((i % 8) * 16, 16)] = jnp.zeros((16,), jnp.float32)
            return 0
        lax.fori_loop(0, 64 * 8, zb, 0)

        def zc(i, _):
            pltpu.sync_copy(zero_v, acc_sh.at[pl.ds(sid * 640 + i * 64, 64)])
            return 0
        lax.fori_loop(0, 10, zc, 0)
        plsc.subcore_barrier()

        if split_edges:
            base0 = (cid * 16 + sid) * (EPAD // NW)
            nchunk = (EPAD // NW) // C
            tbl_off = 0
        else:
            base0 = sid * (EPAD // 16)
            nchunk = (EPAD // 16) // C
            tbl_off = cid * NPAD

        def body(j, _):
            eb = base0 + j * C
            pltpu.sync_copy(u_hbm.at[pl.ds(eb, C)], uraw_v)
            pltpu.sync_copy(v_hbm.at[pl.ds(eb, C)], vraw_v)

            def addoff(k, _):
                ug_v[pl.ds(k * 16, 16)] = uraw_v[pl.ds(k * 16, 16)] + tbl_off
                vg_v[pl.ds(k * 16, 16)] = vraw_v[pl.ds(k * 16, 16)] + tbl_off
                return 0
            lax.fori_loop(0, C // 16, addoff, 0)

            cpA = pltpu.async_copy(y_hbm.at[ug_v], rowsA_v, semA)
            cpB = pltpu.async_copy(y_hbm.at[vg_v], rowsB_v, semB)
            cpA.wait()
            cpB.wait()
            pltpu.sync_copy(rowsA_v, acc_sh.at[vraw_v], add=True)
            pltpu.sync_copy(rowsB_v, acc_sh.at[uraw_v], add=True)
            return 0

        lax.fori_loop(0, nchunk, body, 0)
        plsc.subcore_barrier()

        def wb(i, _):
            r0 = sid * 640 + i * 64
            pltpu.sync_copy(acc_sh.at[pl.ds(r0, 64)],
                            out_hbm.at[cid, pl.ds(r0, 64)])
            return 0
        lax.fori_loop(0, 10, wb, 0)

    return msg_k


_msg_colsplit = _make_msg(False)
_msg_edgesplit = _make_msg(True)


def _select(hg, feats):
    E_, s = hg.shape
    sq = jnp.sum(feats * feats, axis=-1)
    gram = jnp.einsum('esd,etd->est', feats, feats)
    d2 = sq[:, :, None] + sq[:, None, :] - 2.0 * gram
    amax = jnp.argmax(d2.reshape(E_, s * s), axis=1)
    u = jnp.take_along_axis(hg, (amax // s)[:, None], axis=1)[:, 0]
    v = jnp.take_along_axis(hg, (amax % s)[:, None], axis=1)[:, 0]
    ones = jnp.ones((2 * E_,), jnp.float32)
    deg = jax.ops.segment_sum(ones, jnp.concatenate([u, v]),
                              num_segments=N) + 1.0
    dinv = 1.0 / jnp.sqrt(deg)
    return u, v, dinv


_PAD_IDS = None


def _pad_uv(u, v):
    global _PAD_IDS
    if _PAD_IDS is None:
        import numpy as _np
        _PAD_IDS = jnp.asarray(N + (_np.arange(EPAD - E) % (NPAD - N)),
                               dtype=jnp.int32)
    return (jnp.concatenate([u, _PAD_IDS]), jnp.concatenate([v, _PAD_IDS]))


def _mid_body(x_ref, dinv_ref, y_ref, *, cb):
    dv = dinv_ref[...]
    X = x_ref[...]
    for c in range(cb):
        y_ref[c, :, :] = dv * X[:, c * 128:(c + 1) * 128]


def _mid(Xpad, dinvp, cb, block_rows=1024):
    g = NPAD // block_rows
    return pl.pallas_call(
        functools.partial(_mid_body, cb=cb),
        grid=(g,),
        in_specs=[
            pl.BlockSpec((block_rows, 128 * cb), lambda i: (i, 0)),
            pl.BlockSpec((block_rows, 1), lambda i: (i, 0)),
        ],
        out_specs=pl.BlockSpec((cb, block_rows, 128), lambda i: (0, i, 0)),
        out_shape=jax.ShapeDtypeStruct((cb, NPAD, 128), jnp.float32),
    )(Xpad, dinvp)


def _fin1_body(acc_ref, x_ref, dinv_ref, o_ref):
    dv = dinv_ref[...]
    X = x_ref[...]
    for c in range(2):
        o_ref[c, :, :] = jax.nn.relu(
            dv * acc_ref[c, :, :] + dv * dv * X[:, c * 128:(c + 1) * 128])


def _finalize1(acc, Xpad, dinvp, block_rows=1024):
    g = NPAD // block_rows
    return pl.pallas_call(
        _fin1_body,
        grid=(g,),
        in_specs=[
            pl.BlockSpec((2, block_rows, 128), lambda i: (0, i, 0)),
            pl.BlockSpec((block_rows, 256), lambda i: (i, 0)),
            pl.BlockSpec((block_rows, 1), lambda i: (i, 0)),
        ],
        out_specs=pl.BlockSpec((2, block_rows, 128), lambda i: (0, i, 0)),
        out_shape=jax.ShapeDtypeStruct((2, NPAD, 128), jnp.float32),
    )(acc, Xpad, dinvp)


def _fin2_body(acc_ref, x_ref, dinv_ref, o_ref):
    dv = dinv_ref[...]
    a = acc_ref[0, :, :] + acc_ref[1, :, :]
    o_ref[...] = dv * a + dv * dv * x_ref[...]


def _finalize2(acc, X2p, dinvp, block_rows=1024):
    g = NPAD // block_rows
    return pl.pallas_call(
        _fin2_body,
        grid=(g,),
        in_specs=[
            pl.BlockSpec((2, block_rows, 128), lambda i: (0, i, 0)),
            pl.BlockSpec((block_rows, 128), lambda i: (i, 0)),
            pl.BlockSpec((block_rows, 1), lambda i: (i, 0)),
        ],
        out_specs=pl.BlockSpec((block_rows, 128), lambda i: (i, 0)),
        out_shape=jax.ShapeDtypeStruct((NPAD, 128), jnp.float32),
    )(acc, X2p, dinvp)


def kernel(x_list, hg, W1, b1, W2, b2):
    hg_flat = hg.reshape(-1)
    hidden = []
    for k in range(x_list.shape[0]):
        xp = jnp.pad(x_list[k], ((0, NPAD - N), (0, 0)))
        Xp = _project(xp, W1, b1)
        feats = _gather_feats_256(Xp, hg_flat).reshape(E, 4, 256)
        u, v, dinv = _select(hg, feats)
        up, vp = _pad_uv(u, v)
        dinvp = jnp.pad(dinv, (0, NPAD - N)).reshape(NPAD, 1)
        Yb = _mid(Xp, dinvp, 2).reshape(2 * NPAD, 128)
        acc = _msg_colsplit(Yb, up, vp)
        hidden.append(_finalize1(acc, Xp, dinvp))
    h = jnp.concatenate(hidden, axis=0)
    hflat = jnp.concatenate([h[0], h[1], h[2], h[3]], axis=1)
    Hp = _project(hflat, W2, b2)
    Hp128 = jnp.pad(Hp, ((0, 0), (0, 64)))
    feats2 = _gather_feats_128(Hp128, hg_flat)[:, :64].reshape(E, 4, 64)
    u2, v2, dinv2 = _select(hg, feats2)
    up2, vp2 = _pad_uv(u2, v2)
    dinvp2 = jnp.pad(dinv2, (0, NPAD - N)).reshape(NPAD, 1)
    Y2 = _mid(Hp128, dinvp2, 1).reshape(NPAD, 128)
    acc2 = _msg_edgesplit(Y2, up2, vp2)
    out = _finalize2(acc2, Hp128, dinvp2)
    return out[:N, :64]


# speedup vs baseline: 4.7688x; 1.1570x over previous
import functools
import jax
import jax.numpy as jnp
from jax import lax
from jax.experimental import pallas as pl
from jax.experimental.pallas import tpu as pltpu, tpu_sc as plsc

N = 10000
NPAD = 10240
E = 40000
EPAD = 40960
NW = 32


def _matmul_body(x_ref, w_ref, b_ref, o_ref):
    o_ref[...] = (
        jnp.dot(x_ref[...], w_ref[...], preferred_element_type=jnp.float32)
        + b_ref[...]
    )


def _project(X, W, b, block_rows=1024):
    Nr, d = X.shape
    h = W.shape[1]
    return pl.pallas_call(
        _matmul_body,
        grid=(Nr // block_rows,),
        in_specs=[
            pl.BlockSpec((block_rows, d), lambda i: (i, 0)),
            pl.BlockSpec((d, h), lambda i: (0, 0)),
            pl.BlockSpec((h,), lambda i: (0,)),
        ],
        out_specs=pl.BlockSpec((block_rows, h), lambda i: (i, 0)),
        out_shape=jax.ShapeDtypeStruct((Nr, h), jnp.float32),
    )(X, W, b)


def _make_gather(n_idx, width, table_rows):
    per_w = ((n_idx + NW - 1) // NW + 127) // 128 * 128
    C = 128
    mesh = plsc.VectorSubcoreMesh(core_axis_name="c", subcore_axis_name="s")

    @functools.partial(
        pl.kernel,
        out_type=jax.ShapeDtypeStruct((n_idx, width), jnp.float32),
        mesh=mesh,
        scratch_types=[
            pltpu.VMEM((C,), jnp.int32),
            pltpu.VMEM((C, width), jnp.float32),
            pltpu.SemaphoreType.DMA,
        ],
    )
    def gather_k(table_hbm, idx_hbm, out_hbm, idx_v, rows_v, sem):
        wid = lax.axis_index("s") * 2 + lax.axis_index("c")
        base = wid * per_w
        ntodo = jnp.clip(n_idx - base, 0, per_w)
        nchunk = ntodo // C

        def body(j, _):
            off = base + j * C
            pltpu.sync_copy(idx_hbm.at[pl.ds(off, C)], idx_v)
            pltpu.async_copy(table_hbm.at[idx_v], rows_v, sem).wait()
            pltpu.sync_copy(rows_v, out_hbm.at[pl.ds(off, C)])
            return 0

        lax.fori_loop(0, nchunk, body, 0)

    return gather_k


_gather_feats_256 = _make_gather(4 * E, 256, NPAD)
_gather_feats_128 = _make_gather(4 * E, 128, NPAD)


def _make_msg(split_edges):
    C = 128
    mesh = plsc.VectorSubcoreMesh(core_axis_name="c", subcore_axis_name="s")

    @functools.partial(
        pl.kernel,
        out_type=jax.ShapeDtypeStruct((2, NPAD, 128), jnp.float32),
        mesh=mesh,
        scratch_types=[
            pltpu.VMEM((C,), jnp.int32),
            pltpu.VMEM((C,), jnp.int32),
            pltpu.VMEM((C,), jnp.int32),
            pltpu.VMEM((C,), jnp.int32),
            pltpu.VMEM((C, 128), jnp.float32),
            pltpu.VMEM((C, 128), jnp.float32),
            pltpu.VMEM((64, 128), jnp.float32),
            pltpu.VMEM_SHARED((NPAD, 128), jnp.float32),
            pltpu.SemaphoreType.DMA,
            pltpu.SemaphoreType.DMA,
        ],
    )
    def msg_k(y_hbm, u_hbm, v_hbm, out_hbm, uraw_v, vraw_v, ug_v, vg_v,
              rowsA_v, rowsB_v, zero_v, acc_sh, semA, semB):
        cid = lax.axis_index("c")
        sid = lax.axis_index("s")

        def zb(i, _):
            zero_v[i // 8, pl.ds((i % 8) * 16, 16)] = jnp.zeros((16,), jnp.float32)
            return 0
        lax.fori_loop(0, 64 * 8, zb, 0)

        def zc(i, _):
            pltpu.sync_copy(zero_v, acc_sh.at[pl.ds(sid * 640 + i * 64, 64)])
            return 0
        lax.fori_loop(0, 10, zc, 0)
        plsc.subcore_barrier()

        if split_edges:
            base0 = (cid * 16 + sid) * (EPAD // NW)
            nchunk = (EPAD // NW) // C
            tbl_off = 0
        else:
            base0 = sid * (EPAD // 16)
            nchunk = (EPAD // 16) // C
            tbl_off = cid * NPAD

        def body(j, _):
            eb = base0 + j * C
            pltpu.sync_copy(u_hbm.at[pl.ds(eb, C)], uraw_v)
            pltpu.sync_copy(v_hbm.at[pl.ds(eb, C)], vraw_v)

            def addoff(k, _):
                ug_v[pl.ds(k * 16, 16)] = uraw_v[pl.ds(k * 16, 16)] + tbl_off
                vg_v[pl.ds(k * 16, 16)] = vraw_v[pl.ds(k * 16, 16)] + tbl_off
                return 0
            lax.fori_loop(0, C // 16, addoff, 0)

            cpA = pltpu.async_copy(y_hbm.at[ug_v], rowsA_v, semA)
            cpB = pltpu.async_copy(y_hbm.at[vg_v], rowsB_v, semB)
            cpA.wait()
            cpB.wait()
            pltpu.sync_copy(rowsA_v, acc_sh.at[vraw_v], add=True)
            pltpu.sync_copy(rowsB_v, acc_sh.at[uraw_v], add=True)
            return 0

        lax.fori_loop(0, nchunk, body, 0)
        plsc.subcore_barrier()

        def wb(i, _):
            r0 = sid * 640 + i * 64
            pltpu.sync_copy(acc_sh.at[pl.ds(r0, 64)],
                            out_hbm.at[cid, pl.ds(r0, 64)])
            return 0
        lax.fori_loop(0, 10, wb, 0)

    return msg_k


_msg_colsplit = _make_msg(False)
_msg_edgesplit = _make_msg(True)


def _make_deg():
    C = 128
    mesh = plsc.VectorSubcoreMesh(core_axis_name="c", subcore_axis_name="s")

    @functools.partial(
        pl.kernel,
        out_type=jax.ShapeDtypeStruct((2, NPAD), jnp.float32),
        mesh=mesh,
        scratch_types=[
            pltpu.VMEM((C,), jnp.int32),
            pltpu.VMEM((C,), jnp.int32),
            pltpu.VMEM((C,), jnp.float32),
            pltpu.VMEM((640,), jnp.float32),
            pltpu.VMEM_SHARED((NPAD,), jnp.float32),
            pltpu.SemaphoreType.DMA,
        ],
    )
    def deg_k(u_hbm, v_hbm, out_hbm, uraw_v, vraw_v, ones_v, zero_v,
              deg_sh, sem):
        cid = lax.axis_index("c")
        sid = lax.axis_index("s")

        def zb(i, _):
            zero_v[pl.ds(i * 16, 16)] = jnp.zeros((16,), jnp.float32)
            return 0
        lax.fori_loop(0, 40, zb, 0)

        def ob(i, _):
            ones_v[pl.ds(i * 16, 16)] = jnp.ones((16,), jnp.float32)
            return 0
        lax.fori_loop(0, C // 16, ob, 0)

        pltpu.sync_copy(zero_v, deg_sh.at[pl.ds(sid * 640, 640)])
        plsc.subcore_barrier()

        base0 = (cid * 16 + sid) * (EPAD // NW)
        nchunk = (EPAD // NW) // C

        def body(j, _):
            eb = base0 + j * C
            pltpu.sync_copy(u_hbm.at[pl.ds(eb, C)], uraw_v)
            pltpu.sync_copy(v_hbm.at[pl.ds(eb, C)], vraw_v)
            pltpu.sync_copy(ones_v, deg_sh.at[uraw_v], add=True)
            pltpu.sync_copy(ones_v, deg_sh.at[vraw_v], add=True)
            return 0

        lax.fori_loop(0, nchunk, body, 0)
        plsc.subcore_barrier()
        pltpu.sync_copy(deg_sh.at[pl.ds(sid * 640, 640)],
                        out_hbm.at[cid, pl.ds(sid * 640, 640)])

    return deg_k


_deg_kernel = _make_deg()


def _select(hg, feats):
    E_, s = hg.shape
    sq = jnp.sum(feats * feats, axis=-1)
    gram = jnp.einsum('esd,etd->est', feats, feats)
    d2 = sq[:, :, None] + sq[:, None, :] - 2.0 * gram
    amax = jnp.argmax(d2.reshape(E_, s * s), axis=1)
    u = jnp.take_along_axis(hg, (amax // s)[:, None], axis=1)[:, 0]
    v = jnp.take_along_axis(hg, (amax % s)[:, None], axis=1)[:, 0]
    return u, v


_PAD_IDS = None


def _pad_uv(u, v):
    global _PAD_IDS
    if _PAD_IDS is None:
        import numpy as _np
        _PAD_IDS = jnp.asarray(N + (_np.arange(EPAD - E) % (NPAD - N)),
                               dtype=jnp.int32)
    return (jnp.concatenate([u, _PAD_IDS]), jnp.concatenate([v, _PAD_IDS]))


def _mid_body(x_ref, dinv_ref, y_ref, *, cb):
    dv = dinv_ref[...]
    X = x_ref[...]
    for c in range(cb):
        y_ref[c, :, :] = dv * X[:, c * 128:(c + 1) * 128]


def _mid(Xpad, dinvp, cb, block_rows=1024):
    g = NPAD // block_rows
    return pl.pallas_call(
        functools.partial(_mid_body, cb=cb),
        grid=(g,),
        in_specs=[
            pl.BlockSpec((block_rows, 128 * cb), lambda i: (i, 0)),
            pl.BlockSpec((block_rows, 1), lambda i: (i, 0)),
        ],
        out_specs=pl.BlockSpec((cb, block_rows, 128), lambda i: (0, i, 0)),
        out_shape=jax.ShapeDtypeStruct((cb, NPAD, 128), jnp.float32),
    )(Xpad, dinvp)


def _fin1_body(acc_ref, x_ref, dinv_ref, o_ref):
    dv = dinv_ref[...]
    X = x_ref[...]
    for c in range(2):
        o_ref[c, :, :] = jax.nn.relu(
            dv * acc_ref[c, :, :] + dv * dv * X[:, c * 128:(c + 1) * 128])


def _finalize1(acc, Xpad, dinvp, block_rows=1024):
    g = NPAD // block_rows
    return pl.pallas_call(
        _fin1_body,
        grid=(g,),
        in_specs=[
            pl.BlockSpec((2, block_rows, 128), lambda i: (0, i, 0)),
            pl.BlockSpec((block_rows, 256), lambda i: (i, 0)),
            pl.BlockSpec((block_rows, 1), lambda i: (i, 0)),
        ],
        out_specs=pl.BlockSpec((2, block_rows, 128), lambda i: (0, i, 0)),
        out_shape=jax.ShapeDtypeStruct((2, NPAD, 128), jnp.float32),
    )(acc, Xpad, dinvp)


def _fin2_body(acc_ref, x_ref, dinv_ref, o_ref):
    dv = dinv_ref[...]
    a = acc_ref[0, :, :] + acc_ref[1, :, :]
    o_ref[...] = dv * a + dv * dv * x_ref[...]


def _finalize2(acc, X2p, dinvp, block_rows=1024):
    g = NPAD // block_rows
    return pl.pallas_call(
        _fin2_body,
        grid=(g,),
        in_specs=[
            pl.BlockSpec((2, block_rows, 128), lambda i: (0, i, 0)),
            pl.BlockSpec((block_rows, 128), lambda i: (i, 0)),
            pl.BlockSpec((block_rows, 1), lambda i: (i, 0)),
        ],
        out_specs=pl.BlockSpec((block_rows, 128), lambda i: (i, 0)),
        out_shape=jax.ShapeDtypeStruct((NPAD, 128), jnp.float32),
    )(acc, X2p, dinvp)


def kernel(x_list, hg, W1, b1, W2, b2):
    hg_flat = hg.reshape(-1)
    hidden = []
    for k in range(x_list.shape[0]):
        xp = jnp.pad(x_list[k], ((0, NPAD - N), (0, 0)))
        Xp = _project(xp, W1, b1)
        feats = _gather_feats_256(Xp, hg_flat).reshape(E, 4, 256)
        u, v = _select(hg, feats)
        up, vp = _pad_uv(u, v)
        degp = _deg_kernel(up, vp)
        dinvp = (1.0 / jnp.sqrt(degp[0] + degp[1] + 1.0)).reshape(NPAD, 1)
        Yb = _mid(Xp, dinvp, 2).reshape(2 * NPAD, 128)
        acc = _msg_colsplit(Yb, up, vp)
        hidden.append(_finalize1(acc, Xp, dinvp))
    h = jnp.concatenate(hidden, axis=0)
    hflat = jnp.concatenate([h[0], h[1], h[2], h[3]], axis=1)
    Hp = _project(hflat, W2, b2)
    Hp128 = jnp.pad(Hp, ((0, 0), (0, 64)))
    feats2 = _gather_feats_128(Hp128, hg_flat)[:, :64].reshape(E, 4, 64)
    u2, v2 = _select(hg, feats2)
    up2, vp2 = _pad_uv(u2, v2)
    degp2 = _deg_kernel(up2, vp2)
    dinvp2 = (1.0 / jnp.sqrt(degp2[0] + degp2[1] + 1.0)).reshape(NPAD, 1)
    Y2 = _mid(Hp128, dinvp2, 1).reshape(NPAD, 128)
    acc2 = _msg_edgesplit(Y2, up2, vp2)
    out = _finalize2(acc2, Hp128, dinvp2)
    return out[:N, :64]


# speedup vs baseline: 4.9303x; 1.0339x over previous
import functools
import jax
import jax.numpy as jnp
from jax import lax
from jax.experimental import pallas as pl
from jax.experimental.pallas import tpu as pltpu, tpu_sc as plsc

N = 10000
NPAD = 10240
E = 40000
EPAD = 40960
NW = 32


def _matmul_body(x_ref, w_ref, b_ref, o_ref):
    o_ref[...] = (
        jnp.dot(x_ref[...], w_ref[...], preferred_element_type=jnp.float32)
        + b_ref[...]
    )


def _project(X, W, b, block_rows=1024):
    Nr, d = X.shape
    h = W.shape[1]
    return pl.pallas_call(
        _matmul_body,
        grid=(Nr // block_rows,),
        in_specs=[
            pl.BlockSpec((block_rows, d), lambda i: (i, 0)),
            pl.BlockSpec((d, h), lambda i: (0, 0)),
            pl.BlockSpec((h,), lambda i: (0,)),
        ],
        out_specs=pl.BlockSpec((block_rows, h), lambda i: (i, 0)),
        out_shape=jax.ShapeDtypeStruct((Nr, h), jnp.float32),
    )(X, W, b)


def _make_gather(n_idx, width, table_rows):
    per_w = ((n_idx + NW - 1) // NW + 127) // 128 * 128
    C = 64
    NB = 4
    mesh = plsc.VectorSubcoreMesh(core_axis_name="c", subcore_axis_name="s")

    @functools.partial(
        pl.kernel,
        out_type=jax.ShapeDtypeStruct((n_idx, width), jnp.float32),
        mesh=mesh,
        scratch_types=(
            [pltpu.VMEM((C,), jnp.int32) for _ in range(NB)]
            + [pltpu.VMEM((C, width), jnp.float32) for _ in range(NB)]
            + [pltpu.SemaphoreType.DMA for _ in range(2 * NB)]
        ),
    )
    def gather_k(table_hbm, idx_hbm, out_hbm, *scr):
        idxs = scr[:NB]
        rows = scr[NB:2 * NB]
        semg = scr[2 * NB:3 * NB]
        semw = scr[3 * NB:4 * NB]
        wid = lax.axis_index("s") * 2 + lax.axis_index("c")
        base = wid * per_w
        ntodo = jnp.clip(n_idx - base, 0, per_w)
        ngroup = ntodo // (C * NB)

        def grp(g, _):
            offs = [base + (g * NB + b) * C for b in range(NB)]
            for b in range(NB):
                @pl.when(g > 0)
                def _drain(b=b, off=offs[b]):
                    pltpu.make_async_copy(
                        rows[b], out_hbm.at[pl.ds(off, C)], semw[b]).wait()
                pltpu.sync_copy(idx_hbm.at[pl.ds(offs[b], C)], idxs[b])
                pltpu.async_copy(table_hbm.at[idxs[b]], rows[b], semg[b])
            for b in range(NB):
                pltpu.make_async_copy(
                    table_hbm.at[idxs[b]], rows[b], semg[b]).wait()
                pltpu.async_copy(rows[b], out_hbm.at[pl.ds(offs[b], C)],
                                 semw[b])
            return 0

        lax.fori_loop(0, ngroup, grp, 0)
        for b in range(NB):
            @pl.when(ngroup > 0)
            def _final(b=b):
                pltpu.make_async_copy(
                    rows[b], out_hbm.at[pl.ds(base, C)], semw[b]).wait()

    return gather_k


_gather_feats_256 = _make_gather(4 * E, 256, NPAD)
_gather_feats_128 = _make_gather(4 * E, 128, NPAD)


def _make_msg(split_edges):
    C = 128
    mesh = plsc.VectorSubcoreMesh(core_axis_name="c", subcore_axis_name="s")

    @functools.partial(
        pl.kernel,
        out_type=jax.ShapeDtypeStruct((2, NPAD, 128), jnp.float32),
        mesh=mesh,
        scratch_types=[
            pltpu.VMEM((C,), jnp.int32),
            pltpu.VMEM((C,), jnp.int32),
            pltpu.VMEM((C,), jnp.int32),
            pltpu.VMEM((C,), jnp.int32),
            pltpu.VMEM((C, 128), jnp.float32),
            pltpu.VMEM((C, 128), jnp.float32),
            pltpu.VMEM((64, 128), jnp.float32),
            pltpu.VMEM_SHARED((NPAD, 128), jnp.float32),
            pltpu.SemaphoreType.DMA,
            pltpu.SemaphoreType.DMA,
        ],
    )
    def msg_k(y_hbm, u_hbm, v_hbm, out_hbm, uraw_v, vraw_v, ug_v, vg_v,
              rowsA_v, rowsB_v, zero_v, acc_sh, semA, semB):
        cid = lax.axis_index("c")
        sid = lax.axis_index("s")

        def zb(i, _):
            zero_v[i // 8, pl.ds((i % 8) * 16, 16)] = jnp.zeros((16,), jnp.float32)
            return 0
        lax.fori_loop(0, 64 * 8, zb, 0)

        def zc(i, _):
            pltpu.sync_copy(zero_v, acc_sh.at[pl.ds(sid * 640 + i * 64, 64)])
            return 0
        lax.fori_loop(0, 10, zc, 0)
        plsc.subcore_barrier()

        if split_edges:
            base0 = (cid * 16 + sid) * (EPAD // NW)
            nchunk = (EPAD // NW) // C
            tbl_off = 0
        else:
            base0 = sid * (EPAD // 16)
            nchunk = (EPAD // 16) // C
            tbl_off = cid * NPAD

        def body(j, _):
            eb = base0 + j * C
            pltpu.sync_copy(u_hbm.at[pl.ds(eb, C)], uraw_v)
            pltpu.sync_copy(v_hbm.at[pl.ds(eb, C)], vraw_v)

            def addoff(k, _):
                ug_v[pl.ds(k * 16, 16)] = uraw_v[pl.ds(k * 16, 16)] + tbl_off
                vg_v[pl.ds(k * 16, 16)] = vraw_v[pl.ds(k * 16, 16)] + tbl_off
                return 0
            lax.fori_loop(0, C // 16, addoff, 0)

            cpA = pltpu.async_copy(y_hbm.at[ug_v], rowsA_v, semA)
            cpB = pltpu.async_copy(y_hbm.at[vg_v], rowsB_v, semB)
            cpA.wait()
            cpB.wait()
            pltpu.sync_copy(rowsA_v, acc_sh.at[vraw_v], add=True)
            pltpu.sync_copy(rowsB_v, acc_sh.at[uraw_v], add=True)
            return 0

        lax.fori_loop(0, nchunk, body, 0)
        plsc.subcore_barrier()

        def wb(i, _):
            r0 = sid * 640 + i * 64
            pltpu.sync_copy(acc_sh.at[pl.ds(r0, 64)],
                            out_hbm.at[cid, pl.ds(r0, 64)])
            return 0
        lax.fori_loop(0, 10, wb, 0)

    return msg_k


_msg_colsplit = _make_msg(False)
_msg_edgesplit = _make_msg(True)


def _make_deg():
    C = 128
    mesh = plsc.VectorSubcoreMesh(core_axis_name="c", subcore_axis_name="s")

    @functools.partial(
        pl.kernel,
        out_type=jax.ShapeDtypeStruct((2, NPAD), jnp.float32),
        mesh=mesh,
        scratch_types=[
            pltpu.VMEM((C,), jnp.int32),
            pltpu.VMEM((C,), jnp.int32),
            pltpu.VMEM((C,), jnp.float32),
            pltpu.VMEM((640,), jnp.float32),
            pltpu.VMEM_SHARED((NPAD,), jnp.float32),
            pltpu.SemaphoreType.DMA,
        ],
    )
    def deg_k(u_hbm, v_hbm, out_hbm, uraw_v, vraw_v, ones_v, zero_v,
              deg_sh, sem):
        cid = lax.axis_index("c")
        sid = lax.axis_index("s")

        def zb(i, _):
            zero_v[pl.ds(i * 16, 16)] = jnp.zeros((16,), jnp.float32)
            return 0
        lax.fori_loop(0, 40, zb, 0)

        def ob(i, _):
            ones_v[pl.ds(i * 16, 16)] = jnp.ones((16,), jnp.float32)
            return 0
        lax.fori_loop(0, C // 16, ob, 0)

        pltpu.sync_copy(zero_v, deg_sh.at[pl.ds(sid * 640, 640)])
        plsc.subcore_barrier()

        base0 = (cid * 16 + sid) * (EPAD // NW)
        nchunk = (EPAD // NW) // C

        def body(j, _):
            eb = base0 + j * C
            pltpu.sync_copy(u_hbm.at[pl.ds(eb, C)], uraw_v)
            pltpu.sync_copy(v_hbm.at[pl.ds(eb, C)], vraw_v)
            pltpu.sync_copy(ones_v, deg_sh.at[uraw_v], add=True)
            pltpu.sync_copy(ones_v, deg_sh.at[vraw_v], add=True)
            return 0

        lax.fori_loop(0, nchunk, body, 0)
        plsc.subcore_barrier()
        pltpu.sync_copy(deg_sh.at[pl.ds(sid * 640, 640)],
                        out_hbm.at[cid, pl.ds(sid * 640, 640)])

    return deg_k


_deg_kernel = _make_deg()


def _select(hg, feats):
    E_, s = hg.shape
    sq = jnp.sum(feats * feats, axis=-1)
    gram = jnp.einsum('esd,etd->est', feats, feats)
    d2 = sq[:, :, None] + sq[:, None, :] - 2.0 * gram
    amax = jnp.argmax(d2.reshape(E_, s * s), axis=1)
    u = jnp.take_along_axis(hg, (amax // s)[:, None], axis=1)[:, 0]
    v = jnp.take_along_axis(hg, (amax % s)[:, None], axis=1)[:, 0]
    return u, v


_PAD_IDS = None


def _pad_uv(u, v):
    global _PAD_IDS
    if _PAD_IDS is None:
        import numpy as _np
        _PAD_IDS = jnp.asarray(N + (_np.arange(EPAD - E) % (NPAD - N)),
                               dtype=jnp.int32)
    return (jnp.concatenate([u, _PAD_IDS]), jnp.concatenate([v, _PAD_IDS]))


def _mid_body(x_ref, dinv_ref, y_ref, *, cb):
    dv = dinv_ref[...]
    X = x_ref[...]
    for c in range(cb):
        y_ref[c, :, :] = dv * X[:, c * 128:(c + 1) * 128]


def _mid(Xpad, dinvp, cb, block_rows=1024):
    g = NPAD // block_rows
    return pl.pallas_call(
        functools.partial(_mid_body, cb=cb),
        grid=(g,),
        in_specs=[
            pl.BlockSpec((block_rows, 128 * cb), lambda i: (i, 0)),
            pl.BlockSpec((block_rows, 1), lambda i: (i, 0)),
        ],
        out_specs=pl.BlockSpec((cb, block_rows, 128), lambda i: (0, i, 0)),
        out_shape=jax.ShapeDtypeStruct((cb, NPAD, 128), jnp.float32),
    )(Xpad, dinvp)


def _fin1_body(acc_ref, x_ref, dinv_ref, o_ref):
    dv = dinv_ref[...]
    X = x_ref[...]
    for c in range(2):
        o_ref[c, :, :] = jax.nn.relu(
            dv * acc_ref[c, :, :] + dv * dv * X[:, c * 128:(c + 1) * 128])


def _finalize1(acc, Xpad, dinvp, block_rows=1024):
    g = NPAD // block_rows
    return pl.pallas_call(
        _fin1_body,
        grid=(g,),
        in_specs=[
            pl.BlockSpec((2, block_rows, 128), lambda i: (0, i, 0)),
            pl.BlockSpec((block_rows, 256), lambda i: (i, 0)),
            pl.BlockSpec((block_rows, 1), lambda i: (i, 0)),
        ],
        out_specs=pl.BlockSpec((2, block_rows, 128), lambda i: (0, i, 0)),
        out_shape=jax.ShapeDtypeStruct((2, NPAD, 128), jnp.float32),
    )(acc, Xpad, dinvp)


def _fin2_body(acc_ref, x_ref, dinv_ref, o_ref):
    dv = dinv_ref[...]
    a = acc_ref[0, :, :] + acc_ref[1, :, :]
    o_ref[...] = dv * a + dv * dv * x_ref[...]


def _finalize2(acc, X2p, dinvp, block_rows=1024):
    g = NPAD // block_rows
    return pl.pallas_call(
        _fin2_body,
        grid=(g,),
        in_specs=[
            pl.BlockSpec((2, block_rows, 128), lambda i: (0, i, 0)),
            pl.BlockSpec((block_rows, 128), lambda i: (i, 0)),
            pl.BlockSpec((block_rows, 1), lambda i: (i, 0)),
        ],
        out_specs=pl.BlockSpec((block_rows, 128), lambda i: (i, 0)),
        out_shape=jax.ShapeDtypeStruct((NPAD, 128), jnp.float32),
    )(acc, X2p, dinvp)


def kernel(x_list, hg, W1, b1, W2, b2):
    hg_flat = hg.reshape(-1)
    hidden = []
    for k in range(x_list.shape[0]):
        xp = jnp.pad(x_list[k], ((0, NPAD - N), (0, 0)))
        Xp = _project(xp, W1, b1)
        feats = _gather_feats_256(Xp, hg_flat).reshape(E, 4, 256)
        u, v = _select(hg, feats)
        up, vp = _pad_uv(u, v)
        degp = _deg_kernel(up, vp)
        dinvp = (1.0 / jnp.sqrt(degp[0] + degp[1] + 1.0)).reshape(NPAD, 1)
        Yb = _mid(Xp, dinvp, 2).reshape(2 * NPAD, 128)
        acc = _msg_colsplit(Yb, up, vp)
        hidden.append(_finalize1(acc, Xp, dinvp))
    h = jnp.concatenate(hidden, axis=0)
    hflat = jnp.concatenate([h[0], h[1], h[2], h[3]], axis=1)
    Hp = _project(hflat, W2, b2)
    Hp128 = jnp.pad(Hp, ((0, 0), (0, 64)))
    feats2 = _gather_feats_128(Hp128, hg_flat)[:, :64].reshape(E, 4, 64)
    u2, v2 = _select(hg, feats2)
    up2, vp2 = _pad_uv(u2, v2)
    degp2 = _deg_kernel(up2, vp2)
    dinvp2 = (1.0 / jnp.sqrt(degp2[0] + degp2[1] + 1.0)).reshape(NPAD, 1)
    Y2 = _mid(Hp128, dinvp2, 1).reshape(NPAD, 128)
    acc2 = _msg_edgesplit(Y2, up2, vp2)
    out = _finalize2(acc2, Hp128, dinvp2)
    return out[:N, :64]


# speedup vs baseline: 5.0614x; 1.0266x over previous
import functools
import jax
import jax.numpy as jnp
from jax import lax
from jax.experimental import pallas as pl
from jax.experimental.pallas import tpu as pltpu, tpu_sc as plsc

N = 10000
NPAD = 10240
E = 40000
EPAD = 40960
NW = 32


def _matmul_body(x_ref, w_ref, b_ref, o_ref):
    o_ref[...] = (
        jnp.dot(x_ref[...], w_ref[...], preferred_element_type=jnp.float32)
        + b_ref[...]
    )


def _project(X, W, b, block_rows=1024):
    Nr, d = X.shape
    h = W.shape[1]
    return pl.pallas_call(
        _matmul_body,
        grid=(Nr // block_rows,),
        in_specs=[
            pl.BlockSpec((block_rows, d), lambda i: (i, 0)),
            pl.BlockSpec((d, h), lambda i: (0, 0)),
            pl.BlockSpec((h,), lambda i: (0,)),
        ],
        out_specs=pl.BlockSpec((block_rows, h), lambda i: (i, 0)),
        out_shape=jax.ShapeDtypeStruct((Nr, h), jnp.float32),
    )(X, W, b)


def _make_gather(n_idx, width, table_rows):
    per_w = ((n_idx + NW - 1) // NW + 127) // 128 * 128
    C = 64
    NB = 4
    mesh = plsc.VectorSubcoreMesh(core_axis_name="c", subcore_axis_name="s")

    @functools.partial(
        pl.kernel,
        out_type=jax.ShapeDtypeStruct((n_idx, width), jnp.float32),
        mesh=mesh,
        scratch_types=(
            [pltpu.VMEM((C,), jnp.int32) for _ in range(NB)]
            + [pltpu.VMEM((C, width), jnp.float32) for _ in range(NB)]
            + [pltpu.SemaphoreType.DMA for _ in range(2 * NB)]
        ),
    )
    def gather_k(table_hbm, idx_hbm, out_hbm, *scr):
        idxs = scr[:NB]
        rows = scr[NB:2 * NB]
        semg = scr[2 * NB:3 * NB]
        semw = scr[3 * NB:4 * NB]
        wid = lax.axis_index("s") * 2 + lax.axis_index("c")
        base = wid * per_w
        ntodo = jnp.clip(n_idx - base, 0, per_w)
        ngroup = ntodo // (C * NB)

        def grp(g, _):
            offs = [base + (g * NB + b) * C for b in range(NB)]
            for b in range(NB):
                @pl.when(g > 0)
                def _drain(b=b, off=offs[b]):
                    pltpu.make_async_copy(
                        rows[b], out_hbm.at[pl.ds(off, C)], semw[b]).wait()
                pltpu.sync_copy(idx_hbm.at[pl.ds(offs[b], C)], idxs[b])
                pltpu.async_copy(table_hbm.at[idxs[b]], rows[b], semg[b])
            for b in range(NB):
                pltpu.make_async_copy(
                    table_hbm.at[idxs[b]], rows[b], semg[b]).wait()
                pltpu.async_copy(rows[b], out_hbm.at[pl.ds(offs[b], C)],
                                 semw[b])
            return 0

        lax.fori_loop(0, ngroup, grp, 0)
        for b in range(NB):
            @pl.when(ngroup > 0)
            def _final(b=b):
                pltpu.make_async_copy(
                    rows[b], out_hbm.at[pl.ds(base, C)], semw[b]).wait()

    return gather_k


_gather_feats_256 = _make_gather(4 * E, 256, NPAD)
_gather_feats_128 = _make_gather(4 * E, 128, NPAD)


def _make_msg(split_edges):
    C = 64
    mesh = plsc.VectorSubcoreMesh(core_axis_name="c", subcore_axis_name="s")

    @functools.partial(
        pl.kernel,
        out_type=jax.ShapeDtypeStruct((2, NPAD, 128), jnp.float32),
        mesh=mesh,
        scratch_types=(
            [pltpu.VMEM((C,), jnp.int32) for _ in range(8)]
            + [pltpu.VMEM((C, 128), jnp.float32) for _ in range(4)]
            + [pltpu.VMEM((32, 128), jnp.float32),
               pltpu.VMEM_SHARED((NPAD, 128), jnp.float32)]
            + [pltpu.SemaphoreType.DMA for _ in range(4)]
        ),
    )
    def msg_k(y_hbm, u_hbm, v_hbm, out_hbm, *scr):
        uraw = scr[0:2]
        vraw = scr[2:4]
        ug = scr[4:6]
        vg = scr[6:8]
        rowsA = scr[8:10]
        rowsB = scr[10:12]
        zero_v = scr[12]
        acc_sh = scr[13]
        semA = scr[14:16]
        semB = scr[16:18]
        cid = lax.axis_index("c")
        sid = lax.axis_index("s")

        def zb(i, _):
            zero_v[i // 8, pl.ds((i % 8) * 16, 16)] = jnp.zeros((16,), jnp.float32)
            return 0
        lax.fori_loop(0, 32 * 8, zb, 0)

        def zc(i, _):
            pltpu.sync_copy(zero_v, acc_sh.at[pl.ds(sid * 640 + i * 32, 32)])
            return 0
        lax.fori_loop(0, 20, zc, 0)
        plsc.subcore_barrier()

        if split_edges:
            base0 = (cid * 16 + sid) * (EPAD // NW)
            nchunk = (EPAD // NW) // C
            tbl_off = 0
        else:
            base0 = sid * (EPAD // 16)
            nchunk = (EPAD // 16) // C
            tbl_off = cid * NPAD

        def stage(b, j):
            eb = base0 + j * C
            pltpu.sync_copy(u_hbm.at[pl.ds(eb, C)], uraw[b])
            pltpu.sync_copy(v_hbm.at[pl.ds(eb, C)], vraw[b])

            def addoff(k, _):
                ug[b][pl.ds(k * 16, 16)] = uraw[b][pl.ds(k * 16, 16)] + tbl_off
                vg[b][pl.ds(k * 16, 16)] = vraw[b][pl.ds(k * 16, 16)] + tbl_off
                return 0
            lax.fori_loop(0, C // 16, addoff, 0)
            pltpu.async_copy(y_hbm.at[ug[b]], rowsA[b], semA[b])
            pltpu.async_copy(y_hbm.at[vg[b]], rowsB[b], semB[b])

        stage(0, 0)

        def body(j2, _):
            for b in range(2):
                j = j2 * 2 + b

                @pl.when(j + 1 < nchunk)
                def _pre(b=b, j=j):
                    stage(1 - b, j + 1)

                pltpu.make_async_copy(y_hbm.at[ug[b]], rowsA[b],
                                      semA[b]).wait()
                pltpu.make_async_copy(y_hbm.at[vg[b]], rowsB[b],
                                      semB[b]).wait()
                pltpu.sync_copy(rowsA[b], acc_sh.at[vraw[b]], add=True)
                pltpu.sync_copy(rowsB[b], acc_sh.at[uraw[b]], add=True)
            return 0

        lax.fori_loop(0, nchunk // 2, body, 0)
        plsc.subcore_barrier()

        def wb(i, _):
            r0 = sid * 640 + i * 64
            pltpu.sync_copy(acc_sh.at[pl.ds(r0, 64)],
                            out_hbm.at[cid, pl.ds(r0, 64)])
            return 0
        lax.fori_loop(0, 10, wb, 0)

    return msg_k


_msg_colsplit = _make_msg(False)
_msg_edgesplit = _make_msg(True)


def _make_deg():
    C = 128
    mesh = plsc.VectorSubcoreMesh(core_axis_name="c", subcore_axis_name="s")

    @functools.partial(
        pl.kernel,
        out_type=jax.ShapeDtypeStruct((2, NPAD), jnp.float32),
        mesh=mesh,
        scratch_types=[
            pltpu.VMEM((C,), jnp.int32),
            pltpu.VMEM((C,), jnp.int32),
            pltpu.VMEM((C,), jnp.float32),
            pltpu.VMEM((640,), jnp.float32),
            pltpu.VMEM_SHARED((NPAD,), jnp.float32),
            pltpu.SemaphoreType.DMA,
        ],
    )
    def deg_k(u_hbm, v_hbm, out_hbm, uraw_v, vraw_v, ones_v, zero_v,
              deg_sh, sem):
        cid = lax.axis_index("c")
        sid = lax.axis_index("s")

        def zb(i, _):
            zero_v[pl.ds(i * 16, 16)] = jnp.zeros((16,), jnp.float32)
            return 0
        lax.fori_loop(0, 40, zb, 0)

        def ob(i, _):
            ones_v[pl.ds(i * 16, 16)] = jnp.ones((16,), jnp.float32)
            return 0
        lax.fori_loop(0, C // 16, ob, 0)

        pltpu.sync_copy(zero_v, deg_sh.at[pl.ds(sid * 640, 640)])
        plsc.subcore_barrier()

        base0 = (cid * 16 + sid) * (EPAD // NW)
        nchunk = (EPAD // NW) // C

        def body(j, _):
            eb = base0 + j * C
            pltpu.sync_copy(u_hbm.at[pl.ds(eb, C)], uraw_v)
            pltpu.sync_copy(v_hbm.at[pl.ds(eb, C)], vraw_v)
            pltpu.sync_copy(ones_v, deg_sh.at[uraw_v], add=True)
            pltpu.sync_copy(ones_v, deg_sh.at[vraw_v], add=True)
            return 0

        lax.fori_loop(0, nchunk, body, 0)
        plsc.subcore_barrier()
        pltpu.sync_copy(deg_sh.at[pl.ds(sid * 640, 640)],
                        out_hbm.at[cid, pl.ds(sid * 640, 640)])

    return deg_k


_deg_kernel = _make_deg()


def _select(hg, feats):
    E_, s = hg.shape
    sq = jnp.sum(feats * feats, axis=-1)
    gram = jnp.einsum('esd,etd->est', feats, feats)
    d2 = sq[:, :, None] + sq[:, None, :] - 2.0 * gram
    amax = jnp.argmax(d2.reshape(E_, s * s), axis=1)
    u = jnp.take_along_axis(hg, (amax // s)[:, None], axis=1)[:, 0]
    v = jnp.take_along_axis(hg, (amax % s)[:, None], axis=1)[:, 0]
    return u, v


_PAD_IDS = None


def _pad_uv(u, v):
    global _PAD_IDS
    if _PAD_IDS is None:
        import numpy as _np
        _PAD_IDS = jnp.asarray(N + (_np.arange(EPAD - E) % (NPAD - N)),
                               dtype=jnp.int32)
    return (jnp.concatenate([u, _PAD_IDS]), jnp.concatenate([v, _PAD_IDS]))


def _mid_body(x_ref, dinv_ref, y_ref, *, cb):
    dv = dinv_ref[...]
    X = x_ref[...]
    for c in range(cb):
        y_ref[c, :, :] = dv * X[:, c * 128:(c + 1) * 128]


def _mid(Xpad, dinvp, cb, block_rows=1024):
    g = NPAD // block_rows
    return pl.pallas_call(
        functools.partial(_mid_body, cb=cb),
        grid=(g,),
        in_specs=[
            pl.BlockSpec((block_rows, 128 * cb), lambda i: (i, 0)),
            pl.BlockSpec((block_rows, 1), lambda i: (i, 0)),
        ],
        out_specs=pl.BlockSpec((cb, block_rows, 128), lambda i: (0, i, 0)),
        out_shape=jax.ShapeDtypeStruct((cb, NPAD, 128), jnp.float32),
    )(Xpad, dinvp)


def _fin1_body(acc_ref, x_ref, dinv_ref, o_ref):
    dv = dinv_ref[...]
    X = x_ref[...]
    for c in range(2):
        o_ref[c, :, :] = jax.nn.relu(
            dv * acc_ref[c, :, :] + dv * dv * X[:, c * 128:(c + 1) * 128])


def _finalize1(acc, Xpad, dinvp, block_rows=1024):
    g = NPAD // block_rows
    return pl.pallas_call(
        _fin1_body,
        grid=(g,),
        in_specs=[
            pl.BlockSpec((2, block_rows, 128), lambda i: (0, i, 0)),
            pl.BlockSpec((block_rows, 256), lambda i: (i, 0)),
            pl.BlockSpec((block_rows, 1), lambda i: (i, 0)),
        ],
        out_specs=pl.BlockSpec((2, block_rows, 128), lambda i: (0, i, 0)),
        out_shape=jax.ShapeDtypeStruct((2, NPAD, 128), jnp.float32),
    )(acc, Xpad, dinvp)


def _fin2_body(acc_ref, x_ref, dinv_ref, o_ref):
    dv = dinv_ref[...]
    a = acc_ref[0, :, :] + acc_ref[1, :, :]
    o_ref[...] = dv * a + dv * dv * x_ref[...]


def _finalize2(acc, X2p, dinvp, block_rows=1024):
    g = NPAD // block_rows
    return pl.pallas_call(
        _fin2_body,
        grid=(g,),
        in_specs=[
            pl.BlockSpec((2, block_rows, 128), lambda i: (0, i, 0)),
            pl.BlockSpec((block_rows, 128), lambda i: (i, 0)),
            pl.BlockSpec((block_rows, 1), lambda i: (i, 0)),
        ],
        out_specs=pl.BlockSpec((block_rows, 128), lambda i: (i, 0)),
        out_shape=jax.ShapeDtypeStruct((NPAD, 128), jnp.float32),
    )(acc, X2p, dinvp)


def kernel(x_list, hg, W1, b1, W2, b2):
    hg_flat = hg.reshape(-1)
    hidden = []
    for k in range(x_list.shape[0]):
        xp = jnp.pad(x_list[k], ((0, NPAD - N), (0, 0)))
        Xp = _project(xp, W1, b1)
        feats = _gather_feats_256(Xp, hg_flat).reshape(E, 4, 256)
        u, v = _select(hg, feats)
        up, vp = _pad_uv(u, v)
        degp = _deg_kernel(up, vp)
        dinvp = (1.0 / jnp.sqrt(degp[0] + degp[1] + 1.0)).reshape(NPAD, 1)
        Yb = _mid(Xp, dinvp, 2).reshape(2 * NPAD, 128)
        acc = _msg_colsplit(Yb, up, vp)
        hidden.append(_finalize1(acc, Xp, dinvp))
    h = jnp.concatenate(hidden, axis=0)
    hflat = jnp.concatenate([h[0], h[1], h[2], h[3]], axis=1)
    Hp = _project(hflat, W2, b2)
    Hp128 = jnp.pad(Hp, ((0, 0), (0, 64)))
    feats2 = _gather_feats_128(Hp128, hg_flat)[:, :64].reshape(E, 4, 64)
    u2, v2 = _select(hg, feats2)
    up2, vp2 = _pad_uv(u2, v2)
    degp2 = _deg_kernel(up2, vp2)
    dinvp2 = (1.0 / jnp.sqrt(degp2[0] + degp2[1] + 1.0)).reshape(NPAD, 1)
    Y2 = _mid(Hp128, dinvp2, 1).reshape(NPAD, 128)
    acc2 = _msg_edgesplit(Y2, up2, vp2)
    out = _finalize2(acc2, Hp128, dinvp2)
    return out[:N, :64]
